# baseline (speedup 1.0000x reference)
import functools

import jax
import jax.numpy as jnp
from jax import lax
from jax.experimental import pallas as pl
from jax.experimental.pallas import tpu as pltpu
from jax.experimental.pallas import tpu_sc as plsc

_D = 1024
_NC = 2
_NS = 16
_NW = _NC * _NS


@functools.lru_cache(maxsize=None)
def _build_gather(b, s, chunk, nbuf):
    n = b * s
    n_per_w = n // _NW
    w_per_row = s // n_per_w
    nchunk = n_per_w // chunk
    mesh = plsc.VectorSubcoreMesh(core_axis_name="c", subcore_axis_name="s")

    @functools.partial(
        pl.kernel,
        mesh=mesh,
        out_type=jax.ShapeDtypeStruct((n, _D), jnp.float32),
        scratch_types=(
            [pltpu.VMEM((n_per_w,), jnp.int32)]
            + [pltpu.VMEM((chunk, _D), jnp.float32) for _ in range(nbuf)]
            + [pltpu.SemaphoreType.DMA for _ in range(2 * nbuf)]
        ),
    )
    def gather_kernel(idx_hbm, table_hbm, out_hbm, idx_v, *rest):
        bufs = rest[:nbuf]
        gsem = rest[nbuf:2 * nbuf]
        ssem = rest[2 * nbuf:3 * nbuf]
        wid = lax.axis_index("s") * _NC + lax.axis_index("c")
        base = wid * n_per_w
        row = wid // w_per_row
        col = (wid % w_per_row) * n_per_w
        pltpu.sync_copy(idx_hbm.at[row, pl.ds(col, n_per_w)], idx_v)

        def start_gather(c):
            bb = c % nbuf
            return pltpu.async_copy(
                table_hbm.at[idx_v.at[pl.ds(c * chunk, chunk)]], bufs[bb],
                gsem[bb])

        def start_scatter(c):
            bb = c % nbuf
            return pltpu.async_copy(
                bufs[bb], out_hbm.at[pl.ds(base + c * chunk, chunk)],
                ssem[bb])

        gd = [None] * nchunk
        sd = [None] * nchunk
        for c in range(min(nbuf, nchunk)):
            gd[c] = start_gather(c)
        for c in range(nchunk):
            gd[c].wait()
            sd[c] = start_scatter(c)
            nxt = c + nbuf
            if nxt < nchunk:
                sd[c].wait()
                gd[nxt] = start_gather(nxt)
        for c in range(max(0, nchunk - nbuf), nchunk):
            sd[c].wait()

    return gather_kernel


def kernel(x, table):
    b, s = x.shape
    out = _build_gather(b, s, 8, 15)(x, table)
    return out.reshape(b, s, _D)


# speedup vs baseline: 1.0191x; 1.0191x over previous
import functools

import jax
import jax.numpy as jnp
from jax import lax
from jax.experimental import pallas as pl
from jax.experimental.pallas import tpu as pltpu
from jax.experimental.pallas import tpu_sc as plsc

_D = 1024
_NC = 2
_NS = 16
_NW = _NC * _NS


@functools.lru_cache(maxsize=None)
def _build_gather(b, s, chunk, nbuf):
    n = b * s
    n_per_w = n // _NW
    w_per_row = s // n_per_w
    nchunk = n_per_w // chunk
    mesh = plsc.VectorSubcoreMesh(core_axis_name="c", subcore_axis_name="s")

    @functools.partial(
        pl.kernel,
        mesh=mesh,
        out_type=jax.ShapeDtypeStruct((n, _D), jnp.float32),
        scratch_types=(
            [pltpu.VMEM((n_per_w,), jnp.int32)]
            + [pltpu.VMEM((chunk, _D), jnp.float32) for _ in range(nbuf)]
            + [pltpu.SemaphoreType.DMA for _ in range(2 * nbuf)]
        ),
    )
    def gather_kernel(idx_hbm, table_hbm, out_hbm, idx_v, *rest):
        bufs = rest[:nbuf]
        gsem = rest[nbuf:2 * nbuf]
        ssem = rest[2 * nbuf:3 * nbuf]
        wid = lax.axis_index("s") * _NC + lax.axis_index("c")
        base = wid * n_per_w
        row = wid // w_per_row
        col = (wid % w_per_row) * n_per_w
        pltpu.sync_copy(idx_hbm.at[row, pl.ds(col, n_per_w)], idx_v)

        def start_gather(c):
            bb = c % nbuf
            return pltpu.async_copy(
                table_hbm.at[idx_v.at[pl.ds(c * chunk, chunk)]], bufs[bb],
                gsem[bb])

        def start_scatter(c):
            bb = c % nbuf
            return pltpu.async_copy(
                bufs[bb], out_hbm.at[pl.ds(base + c * chunk, chunk)],
                ssem[bb])

        gd = [None] * nchunk
        sd = [None] * nchunk
        for c in range(min(nbuf, nchunk)):
            gd[c] = start_gather(c)
        for c in range(nchunk):
            gd[c].wait()
            sd[c] = start_scatter(c)
            nxt = c + nbuf
            if nxt < nchunk:
                sd[c].wait()
                gd[nxt] = start_gather(nxt)
        for c in range(max(0, nchunk - nbuf), nchunk):
            sd[c].wait()

    return gather_kernel


def kernel(x, table):
    b, s = x.shape
    out = _build_gather(b, s, 16, 6)(x, table)
    return out.reshape(b, s, _D)
